# Initial kernel scaffold; baseline (speedup 1.0000x reference)
import jax
import jax.numpy as jnp
from jax.experimental import pallas as pl


def kernel(x, edge_index, edge_weight, W, b):
    raise NotImplementedError("write your pallas kernel here")


# speedup vs baseline: 4.2480x; 4.2480x over previous
import jax
import jax.numpy as jnp
from jax import lax
from jax.experimental import pallas as pl
from jax.experimental.pallas import tpu as pltpu
from jax.experimental.pallas import tpu_sc as plsc

N = 10000
E = 320000
D = 128
NC = 2
NS = 16
C = 128
TILES = NC * NS
CHUNKS_PER_TILE = 79
E_PAD = TILES * CHUNKS_PER_TILE * C
ROWS_A = 624
ROWS_REM = N - NS * ROWS_A
_BCAST_DNUMS = lax.GatherDimensionNumbers(
    offset_dims=(), collapsed_slice_dims=(0,), start_index_map=(0,))


def _bcast_lane(v16, l):
    idx = jnp.full((16, 1), l, dtype=jnp.int32)
    return lax.gather(v16, idx, _BCAST_DNUMS, (1,),
                      mode=lax.GatherScatterMode.PROMISE_IN_BOUNDS)


def _spmm_body(feat, col2, row2, w2, out, col_t, row_t, w_t, rows_v, acc):
    c = lax.axis_index("c")
    s = lax.axis_index("s")
    tile = s * NC + c

    pltpu.sync_copy(col2.at[tile], col_t)
    pltpu.sync_copy(row2.at[tile], row_t)
    pltpu.sync_copy(w2.at[tile], w_t)

    z = jnp.zeros((16,), jnp.float32)

    def zrow(r, carry):
        for j in range(D // 16):
            rows_v[r, pl.ds(j * 16, 16)] = z
        return carry

    lax.fori_loop(0, C, zrow, 0)
    base_rows = s * ROWS_A
    nfull = ROWS_A // C
    for i in range(nfull):
        pltpu.sync_copy(rows_v, acc.at[pl.ds(base_rows + i * C, C)])
    rem = ROWS_A - nfull * C
    pltpu.sync_copy(rows_v.at[pl.ds(0, rem)],
                    acc.at[pl.ds(base_rows + nfull * C, rem)])

    @pl.when(s == NS - 1)
    def _zero_tail():
        pltpu.sync_copy(rows_v.at[pl.ds(0, ROWS_REM)],
                        acc.at[pl.ds(NS * ROWS_A, ROWS_REM)])

    plsc.subcore_barrier()

    def chunk(k, carry):
        pltpu.sync_copy(feat.at[col_t.at[k]], rows_v)

        def grp(g, carry2):
            wg = w_t[k, pl.ds(g * 16, 16)]
            for l in range(16):
                wl = _bcast_lane(wg, l)
                r = g * 16 + l
                for j in range(D // 16):
                    sl = pl.ds(j * 16, 16)
                    rows_v[r, sl] = rows_v[r, sl] * wl
            return carry2

        lax.fori_loop(0, C // 16, grp, 0)

        pltpu.sync_copy(rows_v, acc.at[row_t.at[k]], add=True)
        return carry

    lax.fori_loop(0, CHUNKS_PER_TILE, chunk, 0)
    plsc.subcore_barrier()

    pltpu.sync_copy(acc.at[pl.ds(base_rows, ROWS_A)],
                    out.at[c, pl.ds(base_rows, ROWS_A)])

    @pl.when(s == NS - 1)
    def _write_tail():
        pltpu.sync_copy(acc.at[pl.ds(NS * ROWS_A, ROWS_REM)],
                        out.at[c, pl.ds(NS * ROWS_A, ROWS_REM)])


def _make_spmm():
    mesh = plsc.VectorSubcoreMesh(core_axis_name="c", subcore_axis_name="s",
                                  num_cores=NC, num_subcores=NS)
    return pl.kernel(
        _spmm_body,
        out_type=jax.ShapeDtypeStruct((NC, N, D), jnp.float32),
        mesh=mesh,
        scratch_types=[
            pltpu.VMEM((CHUNKS_PER_TILE, C), jnp.int32),
            pltpu.VMEM((CHUNKS_PER_TILE, C), jnp.int32),
            pltpu.VMEM((CHUNKS_PER_TILE, C), jnp.float32),
            pltpu.VMEM((C, D), jnp.float32),
            pltpu.VMEM_SHARED((N, D), jnp.float32),
        ],
    )


_ROWS_BLK = 1000


def _add_body(p_ref, o_ref):
    o_ref[...] = p_ref[0] + p_ref[1]


def _h1_add(p):
    return pl.pallas_call(
        _add_body,
        out_shape=jax.ShapeDtypeStruct((N, D), jnp.float32),
        grid=(N // _ROWS_BLK,),
        in_specs=[pl.BlockSpec((NC, _ROWS_BLK, D), lambda i: (0, i, 0))],
        out_specs=pl.BlockSpec((_ROWS_BLK, D), lambda i: (i, 0)),
    )(p)


def _final_body(x_ref, h1_ref, p2_ref, wt_ref, b_ref, o_ref):
    h2 = p2_ref[0] + p2_ref[1]
    acc = jnp.dot(x_ref[...], wt_ref[0:D], preferred_element_type=jnp.float32)
    acc = acc + jnp.dot(h1_ref[...], wt_ref[D:2 * D],
                        preferred_element_type=jnp.float32)
    acc = acc + jnp.dot(h2, wt_ref[2 * D:3 * D],
                        preferred_element_type=jnp.float32)
    o_ref[...] = acc + b_ref[...]


def _final(x, h1, p2, Wt, b2):
    return pl.pallas_call(
        _final_body,
        out_shape=jax.ShapeDtypeStruct((N, D), jnp.float32),
        grid=(N // _ROWS_BLK,),
        in_specs=[
            pl.BlockSpec((_ROWS_BLK, D), lambda i: (i, 0)),
            pl.BlockSpec((_ROWS_BLK, D), lambda i: (i, 0)),
            pl.BlockSpec((NC, _ROWS_BLK, D), lambda i: (0, i, 0)),
            pl.BlockSpec((3 * D, D), lambda i: (0, 0)),
            pl.BlockSpec((1, D), lambda i: (0, 0)),
        ],
        out_specs=pl.BlockSpec((_ROWS_BLK, D), lambda i: (i, 0)),
    )(x, h1, p2, Wt, b2)


def kernel(x, edge_index, edge_weight, W, b):
    pad = E_PAD - E
    shape3 = (TILES, CHUNKS_PER_TILE, C)
    row = jnp.concatenate(
        [edge_index[0], jnp.zeros((pad,), jnp.int32)]).reshape(shape3)
    col = jnp.concatenate(
        [edge_index[1], jnp.zeros((pad,), jnp.int32)]).reshape(shape3)
    w2 = jnp.concatenate(
        [edge_weight, jnp.zeros((pad,), jnp.float32)]).reshape(shape3)
    spmm = _make_spmm()
    p1 = spmm(x, col, row, w2)
    h1 = _h1_add(p1)
    p2 = spmm(h1, col, row, w2)
    return _final(x, h1, p2, W.T, b.reshape(1, D))
